# baseline (speedup 1.0000x reference)
import functools

import jax
import jax.numpy as jnp
import numpy as np
from jax.experimental import pallas as pl
from jax.experimental.pallas import tpu as pltpu

N = 10000
T = 12
F = 128
E = 160000
HID = 256
OUT = 12
NH = 4
DH = F // NH

NB = 200
_INTERPRET = False


def _dense_body(agg_ref, pe_ref, ht_ref, wgcn_ref, wih_ref, whh_ref, bih_ref,
                bhh_ref, win_ref, bin_ref, wout_ref, bout_ref, wff1_ref,
                bff1_ref, wff2_ref, bff2_ref, ln1g_ref, ln1b_ref, ln2g_ref,
                ln2b_ref, lnfg_ref, lnfb_ref, wm1_ref, bm1_ref, wm2_ref,
                bm2_ref, wm3_ref, bm3_ref, out_ref, s_scr, o_scr):
  f32 = jnp.float32
  lane = jax.lax.broadcasted_iota(jnp.int32, (F, NH), 0)
  col = jax.lax.broadcasted_iota(jnp.int32, (F, NH), 1)
  head_mask = (lane // DH == col).astype(f32)
  lane_t = jax.lax.broadcasted_iota(jnp.int32, (NH, F), 1)
  col_t = jax.lax.broadcasted_iota(jnp.int32, (NH, F), 0)
  head_mask_t = (lane_t // DH == col_t).astype(f32)

  agg = agg_ref[...]
  agg2 = agg.reshape(T * NB, F)
  g2 = jnp.dot(agg2, wgcn_ref[...], preferred_element_type=f32)
  gi_all = (jnp.dot(g2, wih_ref[...], preferred_element_type=f32)
            + bih_ref[...]).reshape(T, NB, 3 * F)

  h = ht_ref[...]
  whh = whh_ref[...]
  bhh = bhh_ref[...]
  for t in range(T):
    gi = gi_all[t]
    gh = jnp.dot(h, whh, preferred_element_type=f32) + bhh
    r = jax.nn.sigmoid(gi[:, 0:F] + gh[:, 0:F])
    z = jax.nn.sigmoid(gi[:, F:2 * F] + gh[:, F:2 * F])
    nn_ = jnp.tanh(gi[:, 2 * F:] + r * gh[:, 2 * F:])
    h = (1.0 - z) * nn_ + z * h
    s_scr[t] = h + pe_ref[t]

  def ln(x, g, b):
    m = jnp.mean(x, axis=-1, keepdims=True)
    v = jnp.mean((x - m) ** 2, axis=-1, keepdims=True)
    return (x - m) * jax.lax.rsqrt(v + 1e-5) * g + b

  S = s_scr[...]
  S2 = S.reshape(T * NB, F)
  win = win_ref[...]
  bin_ = bin_ref[...]
  q = (jnp.dot(S2, win[:, 0:F], preferred_element_type=f32)
       + bin_[:, 0:F]).reshape(T, NB, F)
  k = (jnp.dot(S2, win[:, F:2 * F], preferred_element_type=f32)
       + bin_[:, F:2 * F]).reshape(T, NB, F)
  v = (jnp.dot(S2, win[:, 2 * F:], preferred_element_type=f32)
       + bin_[:, 2 * F:]).reshape(T, NB, F)

  scale = 1.0 / np.sqrt(DH)
  for t1 in range(T):
    prod = q[t1][None, :, :] * k
    s = jnp.dot(prod.reshape(T * NB, F), head_mask,
                preferred_element_type=f32).reshape(T, NB, NH) * scale
    mx = jnp.max(s, axis=0, keepdims=True)
    e = jnp.exp(s - mx)
    p = e / jnp.sum(e, axis=0, keepdims=True)
    p_lanes = jnp.dot(p.reshape(T * NB, NH), head_mask_t,
                      preferred_element_type=f32).reshape(T, NB, F)
    o_scr[t1] = jnp.sum(p_lanes * v, axis=0)

  attn2 = (jnp.dot(o_scr[...].reshape(T * NB, F), wout_ref[...],
                   preferred_element_type=f32) + bout_ref[...])
  S2 = ln(S2 + attn2, ln1g_ref[...], ln1b_ref[...])
  ff = jnp.dot(
      jax.nn.relu(jnp.dot(S2, wff1_ref[...], preferred_element_type=f32)
                  + bff1_ref[...]),
      wff2_ref[...], preferred_element_type=f32) + bff2_ref[...]
  S2 = ln(S2 + ff, ln2g_ref[...], ln2b_ref[...])
  S2 = ln(S2, lnfg_ref[...], lnfb_ref[...])
  S3 = S2.reshape(T, NB, F)

  wm1 = wm1_ref[...]
  x1 = bm1_ref[...]
  for t in range(T):
    x1 = x1 + jnp.dot(S3[t], wm1[t * F:(t + 1) * F, :],
                      preferred_element_type=f32)
  x1 = jax.nn.relu(x1)
  x2 = jax.nn.relu(jnp.dot(x1, wm2_ref[...], preferred_element_type=f32)
                   + bm2_ref[...])
  out_ref[...] = (jnp.dot(x2, wm3_ref[...], preferred_element_type=f32)
                  + bm3_ref[...])


def _dense_call(agg_t, pe_t, H_tilde, W_gcn, W_ihT, W_hhT, b_ih2, b_hh2,
                W_inT, b_in2, W_outT, b_out2, W_ff1T, b_ff12, W_ff2T, b_ff22,
                ln1g2, ln1b2, ln2g2, ln2b2, lnfg2, lnfb2,
                W_m1T, b_m12, W_m2T, b_m22, W_m3T, b_m32):
  grid = (N // NB,)

  def tile3(i):
    return (0, i, 0)

  def full2(i):
    return (0, 0)

  in_specs = [
      pl.BlockSpec((T, NB, F), tile3),
      pl.BlockSpec((T, NB, F), tile3),
      pl.BlockSpec((NB, F), lambda i: (i, 0)),
  ]
  for a in (W_gcn, W_ihT, W_hhT, b_ih2, b_hh2, W_inT, b_in2, W_outT, b_out2,
            W_ff1T, b_ff12, W_ff2T, b_ff22, ln1g2, ln1b2, ln2g2, ln2b2,
            lnfg2, lnfb2, W_m1T, b_m12, W_m2T, b_m22, W_m3T, b_m32):
    in_specs.append(pl.BlockSpec(a.shape, full2))

  return pl.pallas_call(
      _dense_body,
      grid=grid,
      in_specs=in_specs,
      out_specs=pl.BlockSpec((NB, OUT), lambda i: (i, 0)),
      out_shape=jax.ShapeDtypeStruct((N, OUT), jnp.float32),
      scratch_shapes=[
          pltpu.VMEM((T, NB, F), jnp.float32),
          pltpu.VMEM((T, NB, F), jnp.float32),
      ],
      compiler_params=pltpu.CompilerParams(
          dimension_semantics=("parallel",)),
      interpret=_INTERPRET,
  )(agg_t, pe_t, H_tilde, W_gcn, W_ihT, W_hhT, b_ih2, b_hh2, W_inT, b_in2,
    W_outT, b_out2, W_ff1T, b_ff12, W_ff2T, b_ff22, ln1g2, ln1b2, ln2g2,
    ln2b2, lnfg2, lnfb2, W_m1T, b_m12, W_m2T, b_m22, W_m3T, b_m32)


@jax.jit
def _run(X, edge_index, edge_weight, H_tilde, pe,
         W_gcn, W_ih, W_hh, b_ih, b_hh,
         W_in, b_in, W_out, b_out, W_ff1, b_ff1, W_ff2, b_ff2,
         ln1_g, ln1_b, ln2_g, ln2_b, lnf_g, lnf_b,
         W_m1, b_m1, W_m2, b_m2, W_m3, b_m3):
  src = edge_index[0]
  dst = edge_index[1]
  Xt = jnp.transpose(X, (1, 0, 2))
  w = edge_weight[:, None]
  agg_t = jax.vmap(
      lambda xt: jax.ops.segment_sum(w * xt[src], dst, num_segments=N))(Xt)
  pe_t = jnp.transpose(pe, (1, 0, 2))

  r2 = lambda b: b.reshape(1, -1)
  return _dense_call(
      agg_t, pe_t, H_tilde, W_gcn, W_ih.T, W_hh.T, r2(b_ih), r2(b_hh),
      W_in.T, r2(b_in), W_out.T, r2(b_out), W_ff1.T, r2(b_ff1), W_ff2.T,
      r2(b_ff2), r2(ln1_g), r2(ln1_b), r2(ln2_g), r2(ln2_b), r2(lnf_g),
      r2(lnf_b), W_m1.T, r2(b_m1), W_m2.T, r2(b_m2), W_m3.T, r2(b_m3))


def kernel(X, edge_index, edge_weight, H_tilde, num_features, pe, W_gcn, W_ih,
           W_hh, b_ih, b_hh, W_in, b_in, W_out, b_out, W_ff1, b_ff1, W_ff2,
           b_ff2, ln1_g, ln1_b, ln2_g, ln2_b, lnf_g, lnf_b, W_m1, b_m1, W_m2,
           b_m2, W_m3, b_m3):
  del num_features
  return _run(X, edge_index, edge_weight, H_tilde, pe,
              W_gcn, W_ih, W_hh, b_ih, b_hh,
              W_in, b_in, W_out, b_out, W_ff1, b_ff1, W_ff2, b_ff2,
              ln1_g, ln1_b, ln2_g, ln2_b, lnf_g, lnf_b,
              W_m1, b_m1, W_m2, b_m2, W_m3, b_m3)


# speedup vs baseline: 5.6799x; 5.6799x over previous
import functools

import jax
import jax.numpy as jnp
import numpy as np
from jax import lax
from jax.experimental import pallas as pl
from jax.experimental.pallas import tpu as pltpu
from jax.experimental.pallas import tpu_sc as plsc

N = 10000
T = 12
F = 128
E = 160000
HID = 256
OUT = 12
NH = 4
DH = F // NH

NB = 200
_INTERPRET = False

NWORK = 32
CK = 128
EW = 40
E_PAD = NWORK * EW * CK
NP = 10240
RPT = NP // 16


def _spmm_body(xr, src3, dst3, w2, zer, out, src_v, dst_v, w_v,
               idx0, idx1, rows0, rows1, acc, sem0, sem1):
  c = lax.axis_index("c")
  s = lax.axis_index("s")
  wid = c * 16 + s
  pltpu.sync_copy(src3.at[wid], src_v)
  pltpu.sync_copy(dst3.at[wid], dst_v)
  pltpu.sync_copy(w2.at[wid], w_v)

  def build_idx(idx_ref, chunk, t):
    for j in range(CK // 16):
      sl = pl.ds(16 * j, 16)
      idx_ref[sl] = src_v[chunk, sl] * T + t

  def scale(rows, chunk):
    def gbody(i, carry):
      wv16 = w_v[pl.ds((chunk * CK + i * 16) * 1, 16)]
      for u in range(16):
        wspl = wv16.at[jnp.full((16,), u, jnp.int32)].get(
            mode="promise_in_bounds")
        r = i * 16 + u
        for j in range(F // 16):
          sl = pl.ds(16 * j, 16)
          rows[r, sl] = rows[r, sl] * wspl
      return carry
    lax.fori_loop(0, CK // 16, gbody, 0)

  def per_t(t, carry):
    pltpu.sync_copy(zer, acc.at[pl.ds(s * RPT, RPT)])
    plsc.subcore_barrier()
    build_idx(idx0, 0, t)
    pltpu.async_copy(xr.at[idx0], rows0, sem0)

    def pair(i, carry2):
      c0 = 2 * i
      build_idx(idx1, c0 + 1, t)
      pltpu.async_copy(xr.at[idx1], rows1, sem1)
      pltpu.make_async_copy(xr.at[idx0], rows0, sem0).wait()
      scale(rows0, c0)
      pltpu.sync_copy(rows0, acc.at[dst_v.at[c0]], add=True)

      @pl.when(i < EW // 2 - 1)
      def _():
        build_idx(idx0, c0 + 2, t)
        pltpu.async_copy(xr.at[idx0], rows0, sem0)

      pltpu.make_async_copy(xr.at[idx1], rows1, sem1).wait()
      scale(rows1, c0 + 1)
      pltpu.sync_copy(rows1, acc.at[dst_v.at[c0 + 1]], add=True)
      return carry2

    lax.fori_loop(0, EW // 2, pair, 0)
    plsc.subcore_barrier()
    pltpu.sync_copy(acc.at[pl.ds(s * RPT, RPT)],
                    out.at[c, t, pl.ds(s * RPT, RPT)])
    return carry

  lax.fori_loop(0, T, per_t, 0)


_spmm_call = functools.partial(
    pl.kernel,
    out_type=jax.ShapeDtypeStruct((2, T, NP, F), jnp.float32),
    mesh=plsc.VectorSubcoreMesh(core_axis_name="c", subcore_axis_name="s"),
    scratch_types=[
        pltpu.VMEM((EW, CK), jnp.int32),
        pltpu.VMEM((EW, CK), jnp.int32),
        pltpu.VMEM((EW * CK,), jnp.float32),
        pltpu.VMEM((CK,), jnp.int32),
        pltpu.VMEM((CK,), jnp.int32),
        pltpu.VMEM((CK, F), jnp.float32),
        pltpu.VMEM((CK, F), jnp.float32),
        pltpu.VMEM_SHARED((NP, F), jnp.float32),
        pltpu.SemaphoreType.DMA,
        pltpu.SemaphoreType.DMA,
    ],
)(_spmm_body)


def _dense_body(agg_ref, agg1_ref, pe_ref, ht_ref, wgcn_ref, wih_ref, whh_ref, bih_ref,
                bhh_ref, win_ref, bin_ref, wout_ref, bout_ref, wff1_ref,
                bff1_ref, wff2_ref, bff2_ref, ln1g_ref, ln1b_ref, ln2g_ref,
                ln2b_ref, lnfg_ref, lnfb_ref, wm1_ref, bm1_ref, wm2_ref,
                bm2_ref, wm3_ref, bm3_ref, out_ref, s_scr, o_scr):
  f32 = jnp.float32
  lane = jax.lax.broadcasted_iota(jnp.int32, (F, NH), 0)
  col = jax.lax.broadcasted_iota(jnp.int32, (F, NH), 1)
  head_mask = (lane // DH == col).astype(f32)
  lane_t = jax.lax.broadcasted_iota(jnp.int32, (NH, F), 1)
  col_t = jax.lax.broadcasted_iota(jnp.int32, (NH, F), 0)
  head_mask_t = (lane_t // DH == col_t).astype(f32)

  agg = agg_ref[...] + agg1_ref[...]
  agg2 = agg.reshape(T * NB, F)
  g2 = jnp.dot(agg2, wgcn_ref[...], preferred_element_type=f32)
  gi_all = (jnp.dot(g2, wih_ref[...], preferred_element_type=f32)
            + bih_ref[...]).reshape(T, NB, 3 * F)

  h = ht_ref[...]
  whh = whh_ref[...]
  bhh = bhh_ref[...]
  for t in range(T):
    gi = gi_all[t]
    gh = jnp.dot(h, whh, preferred_element_type=f32) + bhh
    r = jax.nn.sigmoid(gi[:, 0:F] + gh[:, 0:F])
    z = jax.nn.sigmoid(gi[:, F:2 * F] + gh[:, F:2 * F])
    nn_ = jnp.tanh(gi[:, 2 * F:] + r * gh[:, 2 * F:])
    h = (1.0 - z) * nn_ + z * h
    s_scr[t] = h + pe_ref[t]

  def ln(x, g, b):
    m = jnp.mean(x, axis=-1, keepdims=True)
    v = jnp.mean((x - m) ** 2, axis=-1, keepdims=True)
    return (x - m) * jax.lax.rsqrt(v + 1e-5) * g + b

  S = s_scr[...]
  S2 = S.reshape(T * NB, F)
  win = win_ref[...]
  bin_ = bin_ref[...]
  q = (jnp.dot(S2, win[:, 0:F], preferred_element_type=f32)
       + bin_[:, 0:F]).reshape(T, NB, F)
  k = (jnp.dot(S2, win[:, F:2 * F], preferred_element_type=f32)
       + bin_[:, F:2 * F]).reshape(T, NB, F)
  v = (jnp.dot(S2, win[:, 2 * F:], preferred_element_type=f32)
       + bin_[:, 2 * F:]).reshape(T, NB, F)

  scale = 1.0 / np.sqrt(DH)
  for t1 in range(T):
    prod = q[t1][None, :, :] * k
    s = jnp.dot(prod.reshape(T * NB, F), head_mask,
                preferred_element_type=f32).reshape(T, NB, NH) * scale
    mx = jnp.max(s, axis=0, keepdims=True)
    e = jnp.exp(s - mx)
    p = e / jnp.sum(e, axis=0, keepdims=True)
    p_lanes = jnp.dot(p.reshape(T * NB, NH), head_mask_t,
                      preferred_element_type=f32).reshape(T, NB, F)
    o_scr[t1] = jnp.sum(p_lanes * v, axis=0)

  attn2 = (jnp.dot(o_scr[...].reshape(T * NB, F), wout_ref[...],
                   preferred_element_type=f32) + bout_ref[...])
  S2 = ln(S2 + attn2, ln1g_ref[...], ln1b_ref[...])
  ff = jnp.dot(
      jax.nn.relu(jnp.dot(S2, wff1_ref[...], preferred_element_type=f32)
                  + bff1_ref[...]),
      wff2_ref[...], preferred_element_type=f32) + bff2_ref[...]
  S2 = ln(S2 + ff, ln2g_ref[...], ln2b_ref[...])
  S2 = ln(S2, lnfg_ref[...], lnfb_ref[...])
  S3 = S2.reshape(T, NB, F)

  wm1 = wm1_ref[...]
  x1 = bm1_ref[...]
  for t in range(T):
    x1 = x1 + jnp.dot(S3[t], wm1[t * F:(t + 1) * F, :],
                      preferred_element_type=f32)
  x1 = jax.nn.relu(x1)
  x2 = jax.nn.relu(jnp.dot(x1, wm2_ref[...], preferred_element_type=f32)
                   + bm2_ref[...])
  out_ref[...] = (jnp.dot(x2, wm3_ref[...], preferred_element_type=f32)
                  + bm3_ref[...])


def _dense_call(agg_t, agg1_t, pe_t, H_tilde, W_gcn, W_ihT, W_hhT, b_ih2, b_hh2,
                W_inT, b_in2, W_outT, b_out2, W_ff1T, b_ff12, W_ff2T, b_ff22,
                ln1g2, ln1b2, ln2g2, ln2b2, lnfg2, lnfb2,
                W_m1T, b_m12, W_m2T, b_m22, W_m3T, b_m32):
  grid = (N // NB,)

  def tile3(i):
    return (0, i, 0)

  def full2(i):
    return (0, 0)

  in_specs = [
      pl.BlockSpec((T, NB, F), tile3),
      pl.BlockSpec((T, NB, F), tile3),
      pl.BlockSpec((T, NB, F), tile3),
      pl.BlockSpec((NB, F), lambda i: (i, 0)),
  ]
  for a in (W_gcn, W_ihT, W_hhT, b_ih2, b_hh2, W_inT, b_in2, W_outT, b_out2,
            W_ff1T, b_ff12, W_ff2T, b_ff22, ln1g2, ln1b2, ln2g2, ln2b2,
            lnfg2, lnfb2, W_m1T, b_m12, W_m2T, b_m22, W_m3T, b_m32):
    in_specs.append(pl.BlockSpec(a.shape, full2))

  return pl.pallas_call(
      _dense_body,
      grid=grid,
      in_specs=in_specs,
      out_specs=pl.BlockSpec((NB, OUT), lambda i: (i, 0)),
      out_shape=jax.ShapeDtypeStruct((N, OUT), jnp.float32),
      scratch_shapes=[
          pltpu.VMEM((T, NB, F), jnp.float32),
          pltpu.VMEM((T, NB, F), jnp.float32),
      ],
      compiler_params=pltpu.CompilerParams(
          dimension_semantics=("parallel",)),
      interpret=_INTERPRET,
  )(agg_t, agg1_t, pe_t, H_tilde, W_gcn, W_ihT, W_hhT, b_ih2, b_hh2, W_inT, b_in2,
    W_outT, b_out2, W_ff1T, b_ff12, W_ff2T, b_ff22, ln1g2, ln1b2, ln2g2,
    ln2b2, lnfg2, lnfb2, W_m1T, b_m12, W_m2T, b_m22, W_m3T, b_m32)


@jax.jit
def _run(X, edge_index, edge_weight, H_tilde, pe,
         W_gcn, W_ih, W_hh, b_ih, b_hh,
         W_in, b_in, W_out, b_out, W_ff1, b_ff1, W_ff2, b_ff2,
         ln1_g, ln1_b, ln2_g, ln2_b, lnf_g, lnf_b,
         W_m1, b_m1, W_m2, b_m2, W_m3, b_m3):
  src = edge_index[0]
  dst = edge_index[1]
  pad = E_PAD - E
  srcp = jnp.concatenate([src, jnp.zeros((pad,), jnp.int32)])
  dstp = jnp.concatenate([dst, jnp.zeros((pad,), jnp.int32)])
  wp = jnp.concatenate([edge_weight, jnp.zeros((pad,), jnp.float32)])
  parts = _spmm_call(
      X.reshape(N * T, F),
      srcp.reshape(NWORK, EW, CK),
      dstp.reshape(NWORK, EW, CK),
      wp.reshape(NWORK, EW * CK),
      jnp.zeros((RPT, F), jnp.float32))
  pe_t = jnp.transpose(pe, (1, 0, 2))

  r2 = lambda b: b.reshape(1, -1)
  return _dense_call(
      parts[0], parts[1], pe_t, H_tilde, W_gcn, W_ih.T, W_hh.T, r2(b_ih), r2(b_hh),
      W_in.T, r2(b_in), W_out.T, r2(b_out), W_ff1.T, r2(b_ff1), W_ff2.T,
      r2(b_ff2), r2(ln1_g), r2(ln1_b), r2(ln2_g), r2(ln2_b), r2(lnf_g),
      r2(lnf_b), W_m1.T, r2(b_m1), W_m2.T, r2(b_m2), W_m3.T, r2(b_m3))


def kernel(X, edge_index, edge_weight, H_tilde, num_features, pe, W_gcn, W_ih,
           W_hh, b_ih, b_hh, W_in, b_in, W_out, b_out, W_ff1, b_ff1, W_ff2,
           b_ff2, ln1_g, ln1_b, ln2_g, ln2_b, lnf_g, lnf_b, W_m1, b_m1, W_m2,
           b_m2, W_m3, b_m3):
  del num_features
  return _run(X, edge_index, edge_weight, H_tilde, pe,
              W_gcn, W_ih, W_hh, b_ih, b_hh,
              W_in, b_in, W_out, b_out, W_ff1, b_ff1, W_ff2, b_ff2,
              ln1_g, ln1_b, ln2_g, ln2_b, lnf_g, lnf_b,
              W_m1, b_m1, W_m2, b_m2, W_m3, b_m3)


# speedup vs baseline: 5.7512x; 1.0126x over previous
import functools

import jax
import jax.numpy as jnp
import numpy as np
from jax import lax
from jax.experimental import pallas as pl
from jax.experimental.pallas import tpu as pltpu
from jax.experimental.pallas import tpu_sc as plsc

N = 10000
T = 12
F = 128
E = 160000
HID = 256
OUT = 12
NH = 4
DH = F // NH

NB = 200
_INTERPRET = False

CK = 128
NCH = 1280
NCH_A = NCH + 32
E_PAD = NCH * CK
E_ALLOC = NCH_A * CK
C0_CH = 60
C1_CH = 20
RB = 40
NP = 10240
RPT = NP // 16


def _spmm_body(xr, src2, dst2, w2, zer, out, src_v, dst_v, w_v,
               idx0, idx1, rows0, rows1, acc, sem0, sem1):
  c = lax.axis_index("c")
  s = lax.axis_index("s")
  base_chunk = jnp.where(c == 0, s * C0_CH, 16 * C0_CH + s * C1_CH)
  half = jnp.where(c == 0, C0_CH // 2, C1_CH // 2)
  npairs = jnp.where(c == 0, C0_CH // 4, C1_CH // 4)

  def build_idx(idx_ref, chunk, t):
    for j in range(CK // 16):
      sl = pl.ds(16 * j, 16)
      idx_ref[sl] = src_v[chunk, sl] * T + t

  def scale(rows, chunk):
    def gbody(i, carry):
      wv16 = w_v[chunk, pl.ds(i * 16, 16)]
      for u in range(16):
        wspl = wv16.at[jnp.full((16,), u, jnp.int32)].get(
            mode="promise_in_bounds")
        r = i * 16 + u
        for j in range(F // 16):
          sl = pl.ds(16 * j, 16)
          rows[r, sl] = rows[r, sl] * wspl
      return carry
    lax.fori_loop(0, CK // 16, gbody, 0)

  def per_t(t, carry):
    pltpu.sync_copy(zer, acc.at[pl.ds(s * RPT, RPT)])
    plsc.subcore_barrier()

    for r in range(2):
      cb = base_chunk + r * half
      cb8 = (cb // 8) * 8
      off = cb - cb8
      pltpu.sync_copy(src2.at[pl.ds(cb8, RB)], src_v)
      pltpu.sync_copy(dst2.at[pl.ds(cb8, RB)], dst_v)
      pltpu.sync_copy(w2.at[pl.ds(cb8, RB)], w_v)
      build_idx(idx0, off, t)
      pltpu.async_copy(xr.at[idx0], rows0, sem0)

      def pair(i, carry2):
        c0 = off + 2 * i
        build_idx(idx1, c0 + 1, t)
        pltpu.async_copy(xr.at[idx1], rows1, sem1)
        pltpu.make_async_copy(xr.at[idx0], rows0, sem0).wait()
        scale(rows0, c0)
        pltpu.sync_copy(rows0, acc.at[dst_v.at[c0]], add=True)

        @pl.when(i < npairs - 1)
        def _():
          build_idx(idx0, c0 + 2, t)
          pltpu.async_copy(xr.at[idx0], rows0, sem0)

        pltpu.make_async_copy(xr.at[idx1], rows1, sem1).wait()
        scale(rows1, c0 + 1)
        pltpu.sync_copy(rows1, acc.at[dst_v.at[c0 + 1]], add=True)
        return carry2

      lax.fori_loop(0, npairs, pair, 0)

    plsc.subcore_barrier()
    pltpu.sync_copy(acc.at[pl.ds(s * RPT, RPT)],
                    out.at[c, t, pl.ds(s * RPT, RPT)])
    return carry

  lax.fori_loop(0, T, per_t, 0)


_spmm_call = functools.partial(
    pl.kernel,
    out_type=jax.ShapeDtypeStruct((2, T, NP, F), jnp.float32),
    mesh=plsc.VectorSubcoreMesh(core_axis_name="c", subcore_axis_name="s"),
    scratch_types=[
        pltpu.VMEM((RB, CK), jnp.int32),
        pltpu.VMEM((RB, CK), jnp.int32),
        pltpu.VMEM((RB, CK), jnp.float32),
        pltpu.VMEM((CK,), jnp.int32),
        pltpu.VMEM((CK,), jnp.int32),
        pltpu.VMEM((CK, F), jnp.float32),
        pltpu.VMEM((CK, F), jnp.float32),
        pltpu.VMEM_SHARED((NP, F), jnp.float32),
        pltpu.SemaphoreType.DMA,
        pltpu.SemaphoreType.DMA,
    ],
)(_spmm_body)


def _dense_body(agg_ref, agg1_ref, pe_ref, ht_ref, wgcn_ref, wih_ref, whh_ref, bih_ref,
                bhh_ref, win_ref, bin_ref, wout_ref, bout_ref, wff1_ref,
                bff1_ref, wff2_ref, bff2_ref, ln1g_ref, ln1b_ref, ln2g_ref,
                ln2b_ref, lnfg_ref, lnfb_ref, wm1_ref, bm1_ref, wm2_ref,
                bm2_ref, wm3_ref, bm3_ref, out_ref, s_scr, o_scr):
  f32 = jnp.float32
  lane = jax.lax.broadcasted_iota(jnp.int32, (F, NH), 0)
  col = jax.lax.broadcasted_iota(jnp.int32, (F, NH), 1)
  head_mask = (lane // DH == col).astype(f32)
  lane_t = jax.lax.broadcasted_iota(jnp.int32, (NH, F), 1)
  col_t = jax.lax.broadcasted_iota(jnp.int32, (NH, F), 0)
  head_mask_t = (lane_t // DH == col_t).astype(f32)

  agg = agg_ref[...] + agg1_ref[...]
  agg2 = agg.reshape(T * NB, F)
  g2 = jnp.dot(agg2, wgcn_ref[...], preferred_element_type=f32)
  gi_all = (jnp.dot(g2, wih_ref[...], preferred_element_type=f32)
            + bih_ref[...]).reshape(T, NB, 3 * F)

  h = ht_ref[...]
  whh = whh_ref[...]
  bhh = bhh_ref[...]
  for t in range(T):
    gi = gi_all[t]
    gh = jnp.dot(h, whh, preferred_element_type=f32) + bhh
    r = jax.nn.sigmoid(gi[:, 0:F] + gh[:, 0:F])
    z = jax.nn.sigmoid(gi[:, F:2 * F] + gh[:, F:2 * F])
    nn_ = jnp.tanh(gi[:, 2 * F:] + r * gh[:, 2 * F:])
    h = (1.0 - z) * nn_ + z * h
    s_scr[t] = h + pe_ref[t]

  def ln(x, g, b):
    m = jnp.mean(x, axis=-1, keepdims=True)
    v = jnp.mean((x - m) ** 2, axis=-1, keepdims=True)
    return (x - m) * jax.lax.rsqrt(v + 1e-5) * g + b

  S = s_scr[...]
  S2 = S.reshape(T * NB, F)
  win = win_ref[...]
  bin_ = bin_ref[...]
  q = (jnp.dot(S2, win[:, 0:F], preferred_element_type=f32)
       + bin_[:, 0:F]).reshape(T, NB, F)
  k = (jnp.dot(S2, win[:, F:2 * F], preferred_element_type=f32)
       + bin_[:, F:2 * F]).reshape(T, NB, F)
  v = (jnp.dot(S2, win[:, 2 * F:], preferred_element_type=f32)
       + bin_[:, 2 * F:]).reshape(T, NB, F)

  scale = 1.0 / np.sqrt(DH)
  for t1 in range(T):
    prod = q[t1][None, :, :] * k
    s = jnp.dot(prod.reshape(T * NB, F), head_mask,
                preferred_element_type=f32).reshape(T, NB, NH) * scale
    mx = jnp.max(s, axis=0, keepdims=True)
    e = jnp.exp(s - mx)
    p = e / jnp.sum(e, axis=0, keepdims=True)
    p_lanes = jnp.dot(p.reshape(T * NB, NH), head_mask_t,
                      preferred_element_type=f32).reshape(T, NB, F)
    o_scr[t1] = jnp.sum(p_lanes * v, axis=0)

  attn2 = (jnp.dot(o_scr[...].reshape(T * NB, F), wout_ref[...],
                   preferred_element_type=f32) + bout_ref[...])
  S2 = ln(S2 + attn2, ln1g_ref[...], ln1b_ref[...])
  ff = jnp.dot(
      jax.nn.relu(jnp.dot(S2, wff1_ref[...], preferred_element_type=f32)
                  + bff1_ref[...]),
      wff2_ref[...], preferred_element_type=f32) + bff2_ref[...]
  S2 = ln(S2 + ff, ln2g_ref[...], ln2b_ref[...])
  S2 = ln(S2, lnfg_ref[...], lnfb_ref[...])
  S3 = S2.reshape(T, NB, F)

  wm1 = wm1_ref[...]
  x1 = bm1_ref[...]
  for t in range(T):
    x1 = x1 + jnp.dot(S3[t], wm1[t * F:(t + 1) * F, :],
                      preferred_element_type=f32)
  x1 = jax.nn.relu(x1)
  x2 = jax.nn.relu(jnp.dot(x1, wm2_ref[...], preferred_element_type=f32)
                   + bm2_ref[...])
  out_ref[...] = (jnp.dot(x2, wm3_ref[...], preferred_element_type=f32)
                  + bm3_ref[...])


def _dense_call(agg_t, agg1_t, pe_t, H_tilde, W_gcn, W_ihT, W_hhT, b_ih2, b_hh2,
                W_inT, b_in2, W_outT, b_out2, W_ff1T, b_ff12, W_ff2T, b_ff22,
                ln1g2, ln1b2, ln2g2, ln2b2, lnfg2, lnfb2,
                W_m1T, b_m12, W_m2T, b_m22, W_m3T, b_m32):
  grid = (N // NB,)

  def tile3(i):
    return (0, i, 0)

  def full2(i):
    return (0, 0)

  in_specs = [
      pl.BlockSpec((T, NB, F), tile3),
      pl.BlockSpec((T, NB, F), tile3),
      pl.BlockSpec((T, NB, F), tile3),
      pl.BlockSpec((NB, F), lambda i: (i, 0)),
  ]
  for a in (W_gcn, W_ihT, W_hhT, b_ih2, b_hh2, W_inT, b_in2, W_outT, b_out2,
            W_ff1T, b_ff12, W_ff2T, b_ff22, ln1g2, ln1b2, ln2g2, ln2b2,
            lnfg2, lnfb2, W_m1T, b_m12, W_m2T, b_m22, W_m3T, b_m32):
    in_specs.append(pl.BlockSpec(a.shape, full2))

  return pl.pallas_call(
      _dense_body,
      grid=grid,
      in_specs=in_specs,
      out_specs=pl.BlockSpec((NB, OUT), lambda i: (i, 0)),
      out_shape=jax.ShapeDtypeStruct((N, OUT), jnp.float32),
      scratch_shapes=[
          pltpu.VMEM((T, NB, F), jnp.float32),
          pltpu.VMEM((T, NB, F), jnp.float32),
      ],
      compiler_params=pltpu.CompilerParams(
          dimension_semantics=("parallel",)),
      interpret=_INTERPRET,
  )(agg_t, agg1_t, pe_t, H_tilde, W_gcn, W_ihT, W_hhT, b_ih2, b_hh2, W_inT, b_in2,
    W_outT, b_out2, W_ff1T, b_ff12, W_ff2T, b_ff22, ln1g2, ln1b2, ln2g2,
    ln2b2, lnfg2, lnfb2, W_m1T, b_m12, W_m2T, b_m22, W_m3T, b_m32)


@jax.jit
def _run(X, edge_index, edge_weight, H_tilde, pe,
         W_gcn, W_ih, W_hh, b_ih, b_hh,
         W_in, b_in, W_out, b_out, W_ff1, b_ff1, W_ff2, b_ff2,
         ln1_g, ln1_b, ln2_g, ln2_b, lnf_g, lnf_b,
         W_m1, b_m1, W_m2, b_m2, W_m3, b_m3):
  src = edge_index[0]
  dst = edge_index[1]
  pad = E_ALLOC - E
  srcp = jnp.concatenate([src, jnp.zeros((pad,), jnp.int32)])
  dstp = jnp.concatenate([dst, jnp.zeros((pad,), jnp.int32)])
  wp = jnp.concatenate([edge_weight, jnp.zeros((pad,), jnp.float32)])
  parts = _spmm_call(
      X.reshape(N * T, F),
      srcp.reshape(NCH_A, CK),
      dstp.reshape(NCH_A, CK),
      wp.reshape(NCH_A, CK),
      jnp.zeros((RPT, F), jnp.float32))
  pe_t = jnp.transpose(pe, (1, 0, 2))

  r2 = lambda b: b.reshape(1, -1)
  return _dense_call(
      parts[0], parts[1], pe_t, H_tilde, W_gcn, W_ih.T, W_hh.T, r2(b_ih), r2(b_hh),
      W_in.T, r2(b_in), W_out.T, r2(b_out), W_ff1.T, r2(b_ff1), W_ff2.T,
      r2(b_ff2), r2(ln1_g), r2(ln1_b), r2(ln2_g), r2(ln2_b), r2(lnf_g),
      r2(lnf_b), W_m1.T, r2(b_m1), W_m2.T, r2(b_m2), W_m3.T, r2(b_m3))


def kernel(X, edge_index, edge_weight, H_tilde, num_features, pe, W_gcn, W_ih,
           W_hh, b_ih, b_hh, W_in, b_in, W_out, b_out, W_ff1, b_ff1, W_ff2,
           b_ff2, ln1_g, ln1_b, ln2_g, ln2_b, lnf_g, lnf_b, W_m1, b_m1, W_m2,
           b_m2, W_m3, b_m3):
  del num_features
  return _run(X, edge_index, edge_weight, H_tilde, pe,
              W_gcn, W_ih, W_hh, b_ih, b_hh,
              W_in, b_in, W_out, b_out, W_ff1, b_ff1, W_ff2, b_ff2,
              ln1_g, ln1_b, ln2_g, ln2_b, lnf_g, lnf_b,
              W_m1, b_m1, W_m2, b_m2, W_m3, b_m3)


# speedup vs baseline: 7.2054x; 1.2529x over previous
import functools

import jax
import jax.numpy as jnp
import numpy as np
from jax import lax
from jax.experimental import pallas as pl
from jax.experimental.pallas import tpu as pltpu
from jax.experimental.pallas import tpu_sc as plsc

N = 10000
T = 12
F = 128
E = 160000
HID = 256
OUT = 12
NH = 4
DH = F // NH

NB = 200
_INTERPRET = False

CK = 128
NCH = 1280
NCH_A = NCH + 32
E_PAD = NCH * CK
E_ALLOC = NCH_A * CK
WCH = NCH // 16
RB = WCH // 2
T_SPLIT = 7
NP = 10240
RPT = NP // 16


def _spmm_body(xr, src2, dst2, w2, zer, out, src_v, dst_v, w_v,
               idx0, idx1, rows0, rows1, acc, sem0, sem1):
  c = lax.axis_index("c")
  s = lax.axis_index("s")

  def build_idx(idx_ref, chunk, t):
    for j in range(CK // 16):
      sl = pl.ds(16 * j, 16)
      idx_ref[sl] = src_v[chunk, sl] * T + t

  def scale(rows, chunk):
    def gbody(i, carry):
      wv16 = w_v[chunk, pl.ds(i * 16, 16)]
      for u in range(16):
        wspl = wv16.at[jnp.full((16,), u, jnp.int32)].get(
            mode="promise_in_bounds")
        r = i * 16 + u
        for j in range(F // 16):
          sl = pl.ds(16 * j, 16)
          rows[r, sl] = rows[r, sl] * wspl
      return carry
    lax.fori_loop(0, CK // 16, gbody, 0)

  def per_t(t, carry):
    pltpu.sync_copy(zer, acc.at[pl.ds(s * RPT, RPT)])
    plsc.subcore_barrier()

    for r in range(2):
      cb = s * WCH + r * RB
      pltpu.sync_copy(src2.at[pl.ds(cb, RB)], src_v)
      pltpu.sync_copy(dst2.at[pl.ds(cb, RB)], dst_v)
      pltpu.sync_copy(w2.at[pl.ds(cb, RB)], w_v)
      build_idx(idx0, 0, t)
      pltpu.async_copy(xr.at[idx0], rows0, sem0)

      def pair(i, carry2):
        c0 = 2 * i
        build_idx(idx1, c0 + 1, t)
        pltpu.async_copy(xr.at[idx1], rows1, sem1)
        pltpu.make_async_copy(xr.at[idx0], rows0, sem0).wait()
        scale(rows0, c0)
        pltpu.sync_copy(rows0, acc.at[dst_v.at[c0]], add=True)

        @pl.when(i < RB // 2 - 1)
        def _():
          build_idx(idx0, c0 + 2, t)
          pltpu.async_copy(xr.at[idx0], rows0, sem0)

        pltpu.make_async_copy(xr.at[idx1], rows1, sem1).wait()
        scale(rows1, c0 + 1)
        pltpu.sync_copy(rows1, acc.at[dst_v.at[c0 + 1]], add=True)
        return carry2

      lax.fori_loop(0, RB // 2, pair, 0)

    plsc.subcore_barrier()
    pltpu.sync_copy(acc.at[pl.ds(s * RPT, RPT)],
                    out.at[t, pl.ds(s * RPT, RPT)])
    return carry

  t_lo = jnp.where(c == 0, 0, T_SPLIT)
  t_hi = jnp.where(c == 0, T_SPLIT, T)
  lax.fori_loop(t_lo, t_hi, per_t, 0)


_spmm_call = functools.partial(
    pl.kernel,
    out_type=jax.ShapeDtypeStruct((T, NP, F), jnp.float32),
    mesh=plsc.VectorSubcoreMesh(core_axis_name="c", subcore_axis_name="s"),
    scratch_types=[
        pltpu.VMEM((RB, CK), jnp.int32),
        pltpu.VMEM((RB, CK), jnp.int32),
        pltpu.VMEM((RB, CK), jnp.float32),
        pltpu.VMEM((CK,), jnp.int32),
        pltpu.VMEM((CK,), jnp.int32),
        pltpu.VMEM((CK, F), jnp.float32),
        pltpu.VMEM((CK, F), jnp.float32),
        pltpu.VMEM_SHARED((NP, F), jnp.float32),
        pltpu.SemaphoreType.DMA,
        pltpu.SemaphoreType.DMA,
    ],
)(_spmm_body)


def _dense_body(agg_ref, pe_ref, ht_ref, wgcn_ref, wih_ref, whh_ref, bih_ref,
                bhh_ref, win_ref, bin_ref, wout_ref, bout_ref, wff1_ref,
                bff1_ref, wff2_ref, bff2_ref, ln1g_ref, ln1b_ref, ln2g_ref,
                ln2b_ref, lnfg_ref, lnfb_ref, wm1_ref, bm1_ref, wm2_ref,
                bm2_ref, wm3_ref, bm3_ref, out_ref, s_scr, o_scr):
  f32 = jnp.float32
  lane = jax.lax.broadcasted_iota(jnp.int32, (F, NH), 0)
  col = jax.lax.broadcasted_iota(jnp.int32, (F, NH), 1)
  head_mask = (lane // DH == col).astype(f32)
  lane_t = jax.lax.broadcasted_iota(jnp.int32, (NH, F), 1)
  col_t = jax.lax.broadcasted_iota(jnp.int32, (NH, F), 0)
  head_mask_t = (lane_t // DH == col_t).astype(f32)

  agg = agg_ref[...]
  agg2 = agg.reshape(T * NB, F)
  g2 = jnp.dot(agg2, wgcn_ref[...], preferred_element_type=f32)
  gi_all = (jnp.dot(g2, wih_ref[...], preferred_element_type=f32)
            + bih_ref[...]).reshape(T, NB, 3 * F)

  h = ht_ref[...]
  whh = whh_ref[...]
  bhh = bhh_ref[...]
  for t in range(T):
    gi = gi_all[t]
    gh = jnp.dot(h, whh, preferred_element_type=f32) + bhh
    r = jax.nn.sigmoid(gi[:, 0:F] + gh[:, 0:F])
    z = jax.nn.sigmoid(gi[:, F:2 * F] + gh[:, F:2 * F])
    nn_ = jnp.tanh(gi[:, 2 * F:] + r * gh[:, 2 * F:])
    h = (1.0 - z) * nn_ + z * h
    s_scr[t] = h + pe_ref[t]

  def ln(x, g, b):
    m = jnp.mean(x, axis=-1, keepdims=True)
    v = jnp.mean((x - m) ** 2, axis=-1, keepdims=True)
    return (x - m) * jax.lax.rsqrt(v + 1e-5) * g + b

  S = s_scr[...]
  S2 = S.reshape(T * NB, F)
  win = win_ref[...]
  bin_ = bin_ref[...]
  q = (jnp.dot(S2, win[:, 0:F], preferred_element_type=f32)
       + bin_[:, 0:F]).reshape(T, NB, F)
  k = (jnp.dot(S2, win[:, F:2 * F], preferred_element_type=f32)
       + bin_[:, F:2 * F]).reshape(T, NB, F)
  v = (jnp.dot(S2, win[:, 2 * F:], preferred_element_type=f32)
       + bin_[:, 2 * F:]).reshape(T, NB, F)

  scale = 1.0 / np.sqrt(DH)
  for t1 in range(T):
    prod = q[t1][None, :, :] * k
    s = jnp.dot(prod.reshape(T * NB, F), head_mask,
                preferred_element_type=f32).reshape(T, NB, NH) * scale
    mx = jnp.max(s, axis=0, keepdims=True)
    e = jnp.exp(s - mx)
    p = e / jnp.sum(e, axis=0, keepdims=True)
    p_lanes = jnp.dot(p.reshape(T * NB, NH), head_mask_t,
                      preferred_element_type=f32).reshape(T, NB, F)
    o_scr[t1] = jnp.sum(p_lanes * v, axis=0)

  attn2 = (jnp.dot(o_scr[...].reshape(T * NB, F), wout_ref[...],
                   preferred_element_type=f32) + bout_ref[...])
  S2 = ln(S2 + attn2, ln1g_ref[...], ln1b_ref[...])
  ff = jnp.dot(
      jax.nn.relu(jnp.dot(S2, wff1_ref[...], preferred_element_type=f32)
                  + bff1_ref[...]),
      wff2_ref[...], preferred_element_type=f32) + bff2_ref[...]
  S2 = ln(S2 + ff, ln2g_ref[...], ln2b_ref[...])
  S2 = ln(S2, lnfg_ref[...], lnfb_ref[...])
  S3 = S2.reshape(T, NB, F)

  wm1 = wm1_ref[...]
  x1 = bm1_ref[...]
  for t in range(T):
    x1 = x1 + jnp.dot(S3[t], wm1[t * F:(t + 1) * F, :],
                      preferred_element_type=f32)
  x1 = jax.nn.relu(x1)
  x2 = jax.nn.relu(jnp.dot(x1, wm2_ref[...], preferred_element_type=f32)
                   + bm2_ref[...])
  out_ref[...] = (jnp.dot(x2, wm3_ref[...], preferred_element_type=f32)
                  + bm3_ref[...])


def _dense_call(agg_t, pe_t, H_tilde, W_gcn, W_ihT, W_hhT, b_ih2, b_hh2,
                W_inT, b_in2, W_outT, b_out2, W_ff1T, b_ff12, W_ff2T, b_ff22,
                ln1g2, ln1b2, ln2g2, ln2b2, lnfg2, lnfb2,
                W_m1T, b_m12, W_m2T, b_m22, W_m3T, b_m32):
  grid = (N // NB,)

  def tile3(i):
    return (0, i, 0)

  def full2(i):
    return (0, 0)

  in_specs = [
      pl.BlockSpec((T, NB, F), tile3),
      pl.BlockSpec((T, NB, F), tile3),
      pl.BlockSpec((NB, F), lambda i: (i, 0)),
  ]
  for a in (W_gcn, W_ihT, W_hhT, b_ih2, b_hh2, W_inT, b_in2, W_outT, b_out2,
            W_ff1T, b_ff12, W_ff2T, b_ff22, ln1g2, ln1b2, ln2g2, ln2b2,
            lnfg2, lnfb2, W_m1T, b_m12, W_m2T, b_m22, W_m3T, b_m32):
    in_specs.append(pl.BlockSpec(a.shape, full2))

  return pl.pallas_call(
      _dense_body,
      grid=grid,
      in_specs=in_specs,
      out_specs=pl.BlockSpec((NB, OUT), lambda i: (i, 0)),
      out_shape=jax.ShapeDtypeStruct((N, OUT), jnp.float32),
      scratch_shapes=[
          pltpu.VMEM((T, NB, F), jnp.float32),
          pltpu.VMEM((T, NB, F), jnp.float32),
      ],
      compiler_params=pltpu.CompilerParams(
          dimension_semantics=("parallel",)),
      interpret=_INTERPRET,
  )(agg_t, pe_t, H_tilde, W_gcn, W_ihT, W_hhT, b_ih2, b_hh2, W_inT, b_in2,
    W_outT, b_out2, W_ff1T, b_ff12, W_ff2T, b_ff22, ln1g2, ln1b2, ln2g2,
    ln2b2, lnfg2, lnfb2, W_m1T, b_m12, W_m2T, b_m22, W_m3T, b_m32)


@jax.jit
def _run(X, edge_index, edge_weight, H_tilde, pe,
         W_gcn, W_ih, W_hh, b_ih, b_hh,
         W_in, b_in, W_out, b_out, W_ff1, b_ff1, W_ff2, b_ff2,
         ln1_g, ln1_b, ln2_g, ln2_b, lnf_g, lnf_b,
         W_m1, b_m1, W_m2, b_m2, W_m3, b_m3):
  src = edge_index[0]
  dst = edge_index[1]
  pad = E_ALLOC - E
  srcp = jnp.concatenate([src, jnp.zeros((pad,), jnp.int32)])
  dstp = jnp.concatenate([dst, jnp.zeros((pad,), jnp.int32)])
  wp = jnp.concatenate([edge_weight, jnp.zeros((pad,), jnp.float32)])
  agg_t = _spmm_call(
      X.reshape(N * T, F),
      srcp.reshape(NCH_A, CK),
      dstp.reshape(NCH_A, CK),
      wp.reshape(NCH_A, CK),
      jnp.zeros((RPT, F), jnp.float32))
  pe_t = jnp.transpose(pe, (1, 0, 2))

  r2 = lambda b: b.reshape(1, -1)
  return _dense_call(
      agg_t, pe_t, H_tilde, W_gcn, W_ih.T, W_hh.T, r2(b_ih), r2(b_hh),
      W_in.T, r2(b_in), W_out.T, r2(b_out), W_ff1.T, r2(b_ff1), W_ff2.T,
      r2(b_ff2), r2(ln1_g), r2(ln1_b), r2(ln2_g), r2(ln2_b), r2(lnf_g),
      r2(lnf_b), W_m1.T, r2(b_m1), W_m2.T, r2(b_m2), W_m3.T, r2(b_m3))


def kernel(X, edge_index, edge_weight, H_tilde, num_features, pe, W_gcn, W_ih,
           W_hh, b_ih, b_hh, W_in, b_in, W_out, b_out, W_ff1, b_ff1, W_ff2,
           b_ff2, ln1_g, ln1_b, ln2_g, ln2_b, lnf_g, lnf_b, W_m1, b_m1, W_m2,
           b_m2, W_m3, b_m3):
  del num_features
  return _run(X, edge_index, edge_weight, H_tilde, pe,
              W_gcn, W_ih, W_hh, b_ih, b_hh,
              W_in, b_in, W_out, b_out, W_ff1, b_ff1, W_ff2, b_ff2,
              ln1_g, ln1_b, ln2_g, ln2_b, lnf_g, lnf_b,
              W_m1, b_m1, W_m2, b_m2, W_m3, b_m3)


# speedup vs baseline: 7.3250x; 1.0166x over previous
import functools

import jax
import jax.numpy as jnp
import numpy as np
from jax import lax
from jax.experimental import pallas as pl
from jax.experimental.pallas import tpu as pltpu
from jax.experimental.pallas import tpu_sc as plsc

N = 10000
T = 12
F = 128
E = 160000
HID = 256
OUT = 12
NH = 4
DH = F // NH

NB = 200
_INTERPRET = False

CK = 128
NCH = 1280
NCH_A = NCH + 32
E_PAD = NCH * CK
E_ALLOC = NCH_A * CK
WCH = NCH // 16
RB = WCH // 2
T_SPLIT = 5
NP = 10240
RPT = NP // 16


def _spmm_body(xr, src2, dst2, w2, out, src_v, dst_v, w_v,
               idx0, idx1, rows0, rows1, acc, sem0, sem1):
  c = lax.axis_index("c")
  s = lax.axis_index("s")

  def build_idx(idx_ref, chunk, t):
    for j in range(CK // 16):
      sl = pl.ds(16 * j, 16)
      idx_ref[sl] = src_v[chunk, sl] * T + t

  def scale(rows, chunk):
    def gbody(i, carry):
      wv16 = w_v[chunk, pl.ds(i * 16, 16)]
      for u in range(16):
        wspl = wv16.at[jnp.full((16,), u, jnp.int32)].get(
            mode="promise_in_bounds")
        r = i * 16 + u
        for j in range(F // 16):
          sl = pl.ds(16 * j, 16)
          rows[r, sl] = rows[r, sl] * wspl
      return carry
    lax.fori_loop(0, CK // 16, gbody, 0)

  def per_t(t, carry):
    def zfill(i, zc):
      for j in range(F // 16):
        rows0[i, pl.ds(16 * j, 16)] = jnp.zeros((16,), jnp.float32)
      return zc
    lax.fori_loop(0, CK, zfill, 0)
    for k in range(RPT // CK):
      pltpu.sync_copy(rows0, acc.at[pl.ds(s * RPT + k * CK, CK)])
    plsc.subcore_barrier()

    for r in range(2):
      cb = s * WCH + r * RB
      pltpu.sync_copy(src2.at[pl.ds(cb, RB)], src_v)
      pltpu.sync_copy(dst2.at[pl.ds(cb, RB)], dst_v)
      pltpu.sync_copy(w2.at[pl.ds(cb, RB)], w_v)
      build_idx(idx0, 0, t)
      pltpu.async_copy(xr.at[idx0], rows0, sem0)

      def pair(i, carry2):
        c0 = 2 * i
        build_idx(idx1, c0 + 1, t)
        pltpu.async_copy(xr.at[idx1], rows1, sem1)
        pltpu.make_async_copy(xr.at[idx0], rows0, sem0).wait()
        scale(rows0, c0)
        pltpu.sync_copy(rows0, acc.at[dst_v.at[c0]], add=True)

        @pl.when(i < RB // 2 - 1)
        def _():
          build_idx(idx0, c0 + 2, t)
          pltpu.async_copy(xr.at[idx0], rows0, sem0)

        pltpu.make_async_copy(xr.at[idx1], rows1, sem1).wait()
        scale(rows1, c0 + 1)
        pltpu.sync_copy(rows1, acc.at[dst_v.at[c0 + 1]], add=True)
        return carry2

      lax.fori_loop(0, RB // 2, pair, 0)

    plsc.subcore_barrier()
    pltpu.sync_copy(acc.at[pl.ds(s * RPT, RPT)],
                    out.at[t, pl.ds(s * RPT, RPT)])
    return carry

  t_lo = jnp.where(c == 0, 0, T_SPLIT)
  t_hi = jnp.where(c == 0, T_SPLIT, T)
  lax.fori_loop(t_lo, t_hi, per_t, 0)


_spmm_call = functools.partial(
    pl.kernel,
    out_type=jax.ShapeDtypeStruct((T, NP, F), jnp.float32),
    mesh=plsc.VectorSubcoreMesh(core_axis_name="c", subcore_axis_name="s"),
    scratch_types=[
        pltpu.VMEM((RB, CK), jnp.int32),
        pltpu.VMEM((RB, CK), jnp.int32),
        pltpu.VMEM((RB, CK), jnp.float32),
        pltpu.VMEM((CK,), jnp.int32),
        pltpu.VMEM((CK,), jnp.int32),
        pltpu.VMEM((CK, F), jnp.float32),
        pltpu.VMEM((CK, F), jnp.float32),
        pltpu.VMEM_SHARED((NP, F), jnp.float32),
        pltpu.SemaphoreType.DMA,
        pltpu.SemaphoreType.DMA,
    ],
)(_spmm_body)


def _dense_body(agg_ref, pe_ref, ht_ref, wgcn_ref, wih_ref, whh_ref, bih_ref,
                bhh_ref, win_ref, bin_ref, wout_ref, bout_ref, wff1_ref,
                bff1_ref, wff2_ref, bff2_ref, ln1g_ref, ln1b_ref, ln2g_ref,
                ln2b_ref, lnfg_ref, lnfb_ref, wm1_ref, bm1_ref, wm2_ref,
                bm2_ref, wm3_ref, bm3_ref, out_ref, s_scr, o_scr):
  f32 = jnp.float32
  lane = jax.lax.broadcasted_iota(jnp.int32, (F, NH), 0)
  col = jax.lax.broadcasted_iota(jnp.int32, (F, NH), 1)
  head_mask = (lane // DH == col).astype(f32)
  lane_t = jax.lax.broadcasted_iota(jnp.int32, (NH, F), 1)
  col_t = jax.lax.broadcasted_iota(jnp.int32, (NH, F), 0)
  head_mask_t = (lane_t // DH == col_t).astype(f32)

  agg = agg_ref[...]
  agg2 = agg.reshape(T * NB, F)
  g2 = jnp.dot(agg2, wgcn_ref[...], preferred_element_type=f32)
  gi_all = (jnp.dot(g2, wih_ref[...], preferred_element_type=f32)
            + bih_ref[...]).reshape(T, NB, 3 * F)

  h = ht_ref[...]
  whh = whh_ref[...]
  bhh = bhh_ref[...]
  for t in range(T):
    gi = gi_all[t]
    gh = jnp.dot(h, whh, preferred_element_type=f32) + bhh
    r = jax.nn.sigmoid(gi[:, 0:F] + gh[:, 0:F])
    z = jax.nn.sigmoid(gi[:, F:2 * F] + gh[:, F:2 * F])
    nn_ = jnp.tanh(gi[:, 2 * F:] + r * gh[:, 2 * F:])
    h = (1.0 - z) * nn_ + z * h
    s_scr[t] = h + pe_ref[t]

  def ln(x, g, b):
    m = jnp.mean(x, axis=-1, keepdims=True)
    v = jnp.mean((x - m) ** 2, axis=-1, keepdims=True)
    return (x - m) * jax.lax.rsqrt(v + 1e-5) * g + b

  S = s_scr[...]
  S2 = S.reshape(T * NB, F)
  win = win_ref[...]
  bin_ = bin_ref[...]
  q = (jnp.dot(S2, win[:, 0:F], preferred_element_type=f32)
       + bin_[:, 0:F]).reshape(T, NB, F)
  k = (jnp.dot(S2, win[:, F:2 * F], preferred_element_type=f32)
       + bin_[:, F:2 * F]).reshape(T, NB, F)
  v = (jnp.dot(S2, win[:, 2 * F:], preferred_element_type=f32)
       + bin_[:, 2 * F:]).reshape(T, NB, F)

  scale = 1.0 / np.sqrt(DH)
  for t1 in range(T):
    prod = q[t1][None, :, :] * k
    s = jnp.dot(prod.reshape(T * NB, F), head_mask,
                preferred_element_type=f32).reshape(T, NB, NH) * scale
    mx = jnp.max(s, axis=0, keepdims=True)
    e = jnp.exp(s - mx)
    p = e / jnp.sum(e, axis=0, keepdims=True)
    p_lanes = jnp.dot(p.reshape(T * NB, NH), head_mask_t,
                      preferred_element_type=f32).reshape(T, NB, F)
    o_scr[t1] = jnp.sum(p_lanes * v, axis=0)

  attn2 = (jnp.dot(o_scr[...].reshape(T * NB, F), wout_ref[...],
                   preferred_element_type=f32) + bout_ref[...])
  S2 = ln(S2 + attn2, ln1g_ref[...], ln1b_ref[...])
  ff = jnp.dot(
      jax.nn.relu(jnp.dot(S2, wff1_ref[...], preferred_element_type=f32)
                  + bff1_ref[...]),
      wff2_ref[...], preferred_element_type=f32) + bff2_ref[...]
  S2 = ln(S2 + ff, ln2g_ref[...], ln2b_ref[...])
  S2 = ln(S2, lnfg_ref[...], lnfb_ref[...])
  S3 = S2.reshape(T, NB, F)

  wm1 = wm1_ref[...]
  x1 = bm1_ref[...]
  for t in range(T):
    x1 = x1 + jnp.dot(S3[t], wm1[t * F:(t + 1) * F, :],
                      preferred_element_type=f32)
  x1 = jax.nn.relu(x1)
  x2 = jax.nn.relu(jnp.dot(x1, wm2_ref[...], preferred_element_type=f32)
                   + bm2_ref[...])
  out_ref[...] = (jnp.dot(x2, wm3_ref[...], preferred_element_type=f32)
                  + bm3_ref[...])


def _dense_call(agg_t, pe_t, H_tilde, W_gcn, W_ihT, W_hhT, b_ih2, b_hh2,
                W_inT, b_in2, W_outT, b_out2, W_ff1T, b_ff12, W_ff2T, b_ff22,
                ln1g2, ln1b2, ln2g2, ln2b2, lnfg2, lnfb2,
                W_m1T, b_m12, W_m2T, b_m22, W_m3T, b_m32):
  grid = (N // NB,)

  def tile3(i):
    return (0, i, 0)

  def full2(i):
    return (0, 0)

  in_specs = [
      pl.BlockSpec((T, NB, F), tile3),
      pl.BlockSpec((T, NB, F), tile3),
      pl.BlockSpec((NB, F), lambda i: (i, 0)),
  ]
  for a in (W_gcn, W_ihT, W_hhT, b_ih2, b_hh2, W_inT, b_in2, W_outT, b_out2,
            W_ff1T, b_ff12, W_ff2T, b_ff22, ln1g2, ln1b2, ln2g2, ln2b2,
            lnfg2, lnfb2, W_m1T, b_m12, W_m2T, b_m22, W_m3T, b_m32):
    in_specs.append(pl.BlockSpec(a.shape, full2))

  return pl.pallas_call(
      _dense_body,
      grid=grid,
      in_specs=in_specs,
      out_specs=pl.BlockSpec((NB, OUT), lambda i: (i, 0)),
      out_shape=jax.ShapeDtypeStruct((N, OUT), jnp.float32),
      scratch_shapes=[
          pltpu.VMEM((T, NB, F), jnp.float32),
          pltpu.VMEM((T, NB, F), jnp.float32),
      ],
      compiler_params=pltpu.CompilerParams(
          dimension_semantics=("parallel",)),
      interpret=_INTERPRET,
  )(agg_t, pe_t, H_tilde, W_gcn, W_ihT, W_hhT, b_ih2, b_hh2, W_inT, b_in2,
    W_outT, b_out2, W_ff1T, b_ff12, W_ff2T, b_ff22, ln1g2, ln1b2, ln2g2,
    ln2b2, lnfg2, lnfb2, W_m1T, b_m12, W_m2T, b_m22, W_m3T, b_m32)


@jax.jit
def _run(X, edge_index, edge_weight, H_tilde, pe,
         W_gcn, W_ih, W_hh, b_ih, b_hh,
         W_in, b_in, W_out, b_out, W_ff1, b_ff1, W_ff2, b_ff2,
         ln1_g, ln1_b, ln2_g, ln2_b, lnf_g, lnf_b,
         W_m1, b_m1, W_m2, b_m2, W_m3, b_m3):
  src = edge_index[0]
  dst = edge_index[1]
  pad = E_ALLOC - E
  srcp = jnp.concatenate([src, jnp.zeros((pad,), jnp.int32)])
  dstp = jnp.concatenate([dst, jnp.zeros((pad,), jnp.int32)])
  wp = jnp.concatenate([edge_weight, jnp.zeros((pad,), jnp.float32)])
  agg_t = _spmm_call(
      X.reshape(N * T, F),
      srcp.reshape(NCH_A, CK),
      dstp.reshape(NCH_A, CK),
      wp.reshape(NCH_A, CK))
  pe_t = jnp.transpose(pe, (1, 0, 2))

  r2 = lambda b: b.reshape(1, -1)
  return _dense_call(
      agg_t, pe_t, H_tilde, W_gcn, W_ih.T, W_hh.T, r2(b_ih), r2(b_hh),
      W_in.T, r2(b_in), W_out.T, r2(b_out), W_ff1.T, r2(b_ff1), W_ff2.T,
      r2(b_ff2), r2(ln1_g), r2(ln1_b), r2(ln2_g), r2(ln2_b), r2(lnf_g),
      r2(lnf_b), W_m1.T, r2(b_m1), W_m2.T, r2(b_m2), W_m3.T, r2(b_m3))


def kernel(X, edge_index, edge_weight, H_tilde, num_features, pe, W_gcn, W_ih,
           W_hh, b_ih, b_hh, W_in, b_in, W_out, b_out, W_ff1, b_ff1, W_ff2,
           b_ff2, ln1_g, ln1_b, ln2_g, ln2_b, lnf_g, lnf_b, W_m1, b_m1, W_m2,
           b_m2, W_m3, b_m3):
  del num_features
  return _run(X, edge_index, edge_weight, H_tilde, pe,
              W_gcn, W_ih, W_hh, b_ih, b_hh,
              W_in, b_in, W_out, b_out, W_ff1, b_ff1, W_ff2, b_ff2,
              ln1_g, ln1_b, ln2_g, ln2_b, lnf_g, lnf_b,
              W_m1, b_m1, W_m2, b_m2, W_m3, b_m3)


# speedup vs baseline: 7.9649x; 1.0874x over previous
import functools

import jax
import jax.numpy as jnp
import numpy as np
from jax import lax
from jax.experimental import pallas as pl
from jax.experimental.pallas import tpu as pltpu
from jax.experimental.pallas import tpu_sc as plsc

N = 10000
T = 12
F = 128
E = 160000
HID = 256
OUT = 12
NH = 4
DH = F // NH

NB = 200
_INTERPRET = False

CK = 128
NCH = 1280
NCH_A = NCH + 32
E_PAD = NCH * CK
E_ALLOC = NCH_A * CK
WCH = NCH // 16
RB = WCH // 2
T_SPLIT = 6
NP = 10240
RPT = NP // 16


def _spmm_body(xr, src2, dst2, w2, out, src_v, dst_v, w_v,
               idx0, idx1, rows0, rows1, acc, sem0, sem1):
  c = lax.axis_index("c")
  s = lax.axis_index("s")

  def build_idx(idx_ref, chunk, t):
    for j in range(CK // 16):
      sl = pl.ds(16 * j, 16)
      idx_ref[sl] = src_v[chunk, sl] * T + t

  def scale(rows, chunk):
    def gbody(i, carry):
      wv16 = w_v[chunk, pl.ds(i * 16, 16)]
      for u in range(16):
        wspl = wv16.at[jnp.full((16,), u, jnp.int32)].get(
            mode="promise_in_bounds")
        r = i * 16 + u
        for j in range(F // 16):
          sl = pl.ds(16 * j, 16)
          rows[r, sl] = rows[r, sl] * wspl
      return carry
    lax.fori_loop(0, CK // 16, gbody, 0)

  def per_t(t, carry):
    def zfill(i, zc):
      for j in range(F // 16):
        rows0[i, pl.ds(16 * j, 16)] = jnp.zeros((16,), jnp.float32)
      return zc
    lax.fori_loop(0, CK, zfill, 0)
    for k in range(RPT // CK):
      pltpu.sync_copy(rows0, acc.at[pl.ds(s * RPT + k * CK, CK)])
    plsc.subcore_barrier()

    for r in range(2):
      cb = s * WCH + r * RB
      pltpu.sync_copy(src2.at[pl.ds(cb, RB)], src_v)
      pltpu.sync_copy(dst2.at[pl.ds(cb, RB)], dst_v)
      pltpu.sync_copy(w2.at[pl.ds(cb, RB)], w_v)
      build_idx(idx0, 0, t)
      pltpu.async_copy(xr.at[idx0], rows0, sem0)

      def pair(i, carry2):
        c0 = 2 * i
        build_idx(idx1, c0 + 1, t)
        pltpu.async_copy(xr.at[idx1], rows1, sem1)
        pltpu.make_async_copy(xr.at[idx0], rows0, sem0).wait()
        scale(rows0, c0)
        pltpu.sync_copy(rows0, acc.at[dst_v.at[c0]], add=True)

        @pl.when(i < RB // 2 - 1)
        def _():
          build_idx(idx0, c0 + 2, t)
          pltpu.async_copy(xr.at[idx0], rows0, sem0)

        pltpu.make_async_copy(xr.at[idx1], rows1, sem1).wait()
        scale(rows1, c0 + 1)
        pltpu.sync_copy(rows1, acc.at[dst_v.at[c0 + 1]], add=True)
        return carry2

      lax.fori_loop(0, RB // 2, pair, 0)

    plsc.subcore_barrier()
    pltpu.sync_copy(acc.at[pl.ds(s * RPT, RPT)],
                    out.at[t, pl.ds(s * RPT, RPT)])
    return carry

  t_lo = jnp.where(c == 0, 0, T_SPLIT)
  t_hi = jnp.where(c == 0, T_SPLIT, T)
  lax.fori_loop(t_lo, t_hi, per_t, 0)


_spmm_call = functools.partial(
    pl.kernel,
    out_type=jax.ShapeDtypeStruct((T, NP, F), jnp.float32),
    mesh=plsc.VectorSubcoreMesh(core_axis_name="c", subcore_axis_name="s"),
    scratch_types=[
        pltpu.VMEM((RB, CK), jnp.int32),
        pltpu.VMEM((RB, CK), jnp.int32),
        pltpu.VMEM((RB, CK), jnp.float32),
        pltpu.VMEM((CK,), jnp.int32),
        pltpu.VMEM((CK,), jnp.int32),
        pltpu.VMEM((CK, F), jnp.float32),
        pltpu.VMEM((CK, F), jnp.float32),
        pltpu.VMEM_SHARED((NP, F), jnp.float32),
        pltpu.SemaphoreType.DMA,
        pltpu.SemaphoreType.DMA,
    ],
)(_spmm_body)


def _dense_body(agg_ref, pe_ref, ht_ref, wgcn_ref, wih_ref, whh_ref, bih_ref,
                bhh_ref, win_ref, bin_ref, wout_ref, bout_ref, wff1_ref,
                bff1_ref, wff2_ref, bff2_ref, ln1g_ref, ln1b_ref, ln2g_ref,
                ln2b_ref, lnfg_ref, lnfb_ref, wm1_ref, bm1_ref, wm2_ref,
                bm2_ref, wm3_ref, bm3_ref, out_ref, s_scr, o_scr):
  f32 = jnp.float32
  lane = jax.lax.broadcasted_iota(jnp.int32, (F, NH), 0)
  col = jax.lax.broadcasted_iota(jnp.int32, (F, NH), 1)
  head_mask = (lane // DH == col).astype(f32)
  lane_t = jax.lax.broadcasted_iota(jnp.int32, (NH, F), 1)
  col_t = jax.lax.broadcasted_iota(jnp.int32, (NH, F), 0)
  head_mask_t = (lane_t // DH == col_t).astype(f32)

  agg = agg_ref[...]
  agg2 = agg.reshape(T * NB, F)
  g2 = jnp.dot(agg2, wgcn_ref[...], preferred_element_type=f32)
  gi_all = (jnp.dot(g2, wih_ref[...], preferred_element_type=f32)
            + bih_ref[...]).reshape(T, NB, 3 * F)

  h = ht_ref[...]
  whh = whh_ref[...]
  bhh = bhh_ref[...]
  for t in range(T):
    gi = gi_all[t]
    gh = jnp.dot(h, whh, preferred_element_type=f32) + bhh
    r = jax.nn.sigmoid(gi[:, 0:F] + gh[:, 0:F])
    z = jax.nn.sigmoid(gi[:, F:2 * F] + gh[:, F:2 * F])
    nn_ = jnp.tanh(gi[:, 2 * F:] + r * gh[:, 2 * F:])
    h = (1.0 - z) * nn_ + z * h
    s_scr[t] = h + pe_ref[:, t, :]

  def ln(x, g, b):
    m = jnp.mean(x, axis=-1, keepdims=True)
    v = jnp.mean((x - m) ** 2, axis=-1, keepdims=True)
    return (x - m) * jax.lax.rsqrt(v + 1e-5) * g + b

  S = s_scr[...]
  S2 = S.reshape(T * NB, F)
  win = win_ref[...]
  bin_ = bin_ref[...]
  q = (jnp.dot(S2, win[:, 0:F], preferred_element_type=f32)
       + bin_[:, 0:F]).reshape(T, NB, F)
  k = (jnp.dot(S2, win[:, F:2 * F], preferred_element_type=f32)
       + bin_[:, F:2 * F]).reshape(T, NB, F)
  v = (jnp.dot(S2, win[:, 2 * F:], preferred_element_type=f32)
       + bin_[:, 2 * F:]).reshape(T, NB, F)

  scale = 1.0 / np.sqrt(DH)
  for t1 in range(T):
    prod = q[t1][None, :, :] * k
    s = jnp.dot(prod.reshape(T * NB, F), head_mask,
                preferred_element_type=f32).reshape(T, NB, NH) * scale
    mx = jnp.max(s, axis=0, keepdims=True)
    e = jnp.exp(s - mx)
    p = e / jnp.sum(e, axis=0, keepdims=True)
    p_lanes = jnp.dot(p.reshape(T * NB, NH), head_mask_t,
                      preferred_element_type=f32).reshape(T, NB, F)
    o_scr[t1] = jnp.sum(p_lanes * v, axis=0)

  attn2 = (jnp.dot(o_scr[...].reshape(T * NB, F), wout_ref[...],
                   preferred_element_type=f32) + bout_ref[...])
  S2 = ln(S2 + attn2, ln1g_ref[...], ln1b_ref[...])
  ff = jnp.dot(
      jax.nn.relu(jnp.dot(S2, wff1_ref[...], preferred_element_type=f32)
                  + bff1_ref[...]),
      wff2_ref[...], preferred_element_type=f32) + bff2_ref[...]
  S2 = ln(S2 + ff, ln2g_ref[...], ln2b_ref[...])
  S2 = ln(S2, lnfg_ref[...], lnfb_ref[...])
  S3 = S2.reshape(T, NB, F)

  wm1 = wm1_ref[...]
  x1 = bm1_ref[...]
  for t in range(T):
    x1 = x1 + jnp.dot(S3[t], wm1[t * F:(t + 1) * F, :],
                      preferred_element_type=f32)
  x1 = jax.nn.relu(x1)
  x2 = jax.nn.relu(jnp.dot(x1, wm2_ref[...], preferred_element_type=f32)
                   + bm2_ref[...])
  out_ref[...] = (jnp.dot(x2, wm3_ref[...], preferred_element_type=f32)
                  + bm3_ref[...])


def _dense_call(agg_t, pe_t, H_tilde, W_gcn, W_ihT, W_hhT, b_ih2, b_hh2,
                W_inT, b_in2, W_outT, b_out2, W_ff1T, b_ff12, W_ff2T, b_ff22,
                ln1g2, ln1b2, ln2g2, ln2b2, lnfg2, lnfb2,
                W_m1T, b_m12, W_m2T, b_m22, W_m3T, b_m32):
  grid = (N // NB,)

  def tile3(i):
    return (0, i, 0)

  def full2(i):
    return (0, 0)

  in_specs = [
      pl.BlockSpec((T, NB, F), tile3),
      pl.BlockSpec((NB, T, F), lambda i: (i, 0, 0)),
      pl.BlockSpec((NB, F), lambda i: (i, 0)),
  ]
  for a in (W_gcn, W_ihT, W_hhT, b_ih2, b_hh2, W_inT, b_in2, W_outT, b_out2,
            W_ff1T, b_ff12, W_ff2T, b_ff22, ln1g2, ln1b2, ln2g2, ln2b2,
            lnfg2, lnfb2, W_m1T, b_m12, W_m2T, b_m22, W_m3T, b_m32):
    in_specs.append(pl.BlockSpec(a.shape, full2))

  return pl.pallas_call(
      _dense_body,
      grid=grid,
      in_specs=in_specs,
      out_specs=pl.BlockSpec((NB, OUT), lambda i: (i, 0)),
      out_shape=jax.ShapeDtypeStruct((N, OUT), jnp.float32),
      scratch_shapes=[
          pltpu.VMEM((T, NB, F), jnp.float32),
          pltpu.VMEM((T, NB, F), jnp.float32),
      ],
      compiler_params=pltpu.CompilerParams(
          dimension_semantics=("parallel",)),
      interpret=_INTERPRET,
  )(agg_t, pe_t, H_tilde, W_gcn, W_ihT, W_hhT, b_ih2, b_hh2, W_inT, b_in2,
    W_outT, b_out2, W_ff1T, b_ff12, W_ff2T, b_ff22, ln1g2, ln1b2, ln2g2,
    ln2b2, lnfg2, lnfb2, W_m1T, b_m12, W_m2T, b_m22, W_m3T, b_m32)


@jax.jit
def _run(X, edge_index, edge_weight, H_tilde, pe,
         W_gcn, W_ih, W_hh, b_ih, b_hh,
         W_in, b_in, W_out, b_out, W_ff1, b_ff1, W_ff2, b_ff2,
         ln1_g, ln1_b, ln2_g, ln2_b, lnf_g, lnf_b,
         W_m1, b_m1, W_m2, b_m2, W_m3, b_m3):
  src = edge_index[0]
  dst = edge_index[1]
  pad = E_ALLOC - E
  srcp = jnp.concatenate([src, jnp.zeros((pad,), jnp.int32)])
  dstp = jnp.concatenate([dst, jnp.zeros((pad,), jnp.int32)])
  wp = jnp.concatenate([edge_weight, jnp.zeros((pad,), jnp.float32)])
  agg_t = _spmm_call(
      X.reshape(N * T, F),
      srcp.reshape(NCH_A, CK),
      dstp.reshape(NCH_A, CK),
      wp.reshape(NCH_A, CK))

  r2 = lambda b: b.reshape(1, -1)
  return _dense_call(
      agg_t, pe, H_tilde, W_gcn, W_ih.T, W_hh.T, r2(b_ih), r2(b_hh),
      W_in.T, r2(b_in), W_out.T, r2(b_out), W_ff1.T, r2(b_ff1), W_ff2.T,
      r2(b_ff2), r2(ln1_g), r2(ln1_b), r2(ln2_g), r2(ln2_b), r2(lnf_g),
      r2(lnf_b), W_m1.T, r2(b_m1), W_m2.T, r2(b_m2), W_m3.T, r2(b_m3))


def kernel(X, edge_index, edge_weight, H_tilde, num_features, pe, W_gcn, W_ih,
           W_hh, b_ih, b_hh, W_in, b_in, W_out, b_out, W_ff1, b_ff1, W_ff2,
           b_ff2, ln1_g, ln1_b, ln2_g, ln2_b, lnf_g, lnf_b, W_m1, b_m1, W_m2,
           b_m2, W_m3, b_m3):
  del num_features
  return _run(X, edge_index, edge_weight, H_tilde, pe,
              W_gcn, W_ih, W_hh, b_ih, b_hh,
              W_in, b_in, W_out, b_out, W_ff1, b_ff1, W_ff2, b_ff2,
              ln1_g, ln1_b, ln2_g, ln2_b, lnf_g, lnf_b,
              W_m1, b_m1, W_m2, b_m2, W_m3, b_m3)
